# Initial kernel scaffold; baseline (speedup 1.0000x reference)
import jax
import jax.numpy as jnp
from jax.experimental import pallas as pl


def kernel(rec, data, rec_normals, data_normals):
    raise NotImplementedError("write your pallas kernel here")


# speedup vs baseline: 1.2645x; 1.2645x over previous
import functools

import jax
import jax.numpy as jnp
from jax import lax
from jax.experimental import pallas as pl
from jax.experimental.pallas import tpu as pltpu
from jax.experimental.pallas import tpu_sc as plsc

_N = 8192
_M = 8192
_NT = 256


def _argmin_body(rec_ref, data_t_ref, idx_ref):
    rec = rec_ref[...]
    data_t = data_t_ref[...]
    x2 = jnp.sum(rec * rec, axis=1, keepdims=True)
    y2 = jnp.sum(data_t * data_t, axis=0, keepdims=True)
    xy = jnp.dot(rec, data_t, preferred_element_type=jnp.float32)
    d2 = (x2 + y2) - 2.0 * xy
    idx = jnp.argmin(d2, axis=1).astype(jnp.int32)
    idx_ref[...] = idx.reshape(_NT, 1)


def _loss_body(sq_ref, out_ref):
    sq = sq_ref[...]
    out_ref[...] = (jnp.sum(jnp.sqrt(sq)) / _N).reshape(1, 1)


def _sc_residuals(dn_planes, rn_planes, idx):
    info = plsc.get_sparse_core_info()
    nw = info.num_cores * info.num_subcores
    b_per_w = _N // nw
    n_chunks = b_per_w // 16
    mesh = plsc.VectorSubcoreMesh(core_axis_name="c", subcore_axis_name="s")

    @functools.partial(
        pl.kernel,
        out_type=jax.ShapeDtypeStruct((_N,), jnp.float32),
        mesh=mesh,
        compiler_params=pltpu.CompilerParams(needs_layout_passes=False),
        scratch_types=[
            pltpu.VMEM((_M,), jnp.float32),
            pltpu.VMEM((_M,), jnp.float32),
            pltpu.VMEM((_M,), jnp.float32),
            pltpu.VMEM((b_per_w,), jnp.float32),
            pltpu.VMEM((b_per_w,), jnp.float32),
            pltpu.VMEM((b_per_w,), jnp.float32),
            pltpu.VMEM((b_per_w,), jnp.int32),
            pltpu.VMEM((b_per_w,), jnp.float32),
        ],
    )
    def residual_kernel(dnx_h, dny_h, dnz_h, rnx_h, rny_h, rnz_h, idx_h,
                        out_h, dnx_v, dny_v, dnz_v, rnx_v, rny_v, rnz_v,
                        idx_v, out_v):
        wid = lax.axis_index("s") * info.num_cores + lax.axis_index("c")
        base = wid * b_per_w
        pltpu.sync_copy(dnx_h, dnx_v)
        pltpu.sync_copy(dny_h, dny_v)
        pltpu.sync_copy(dnz_h, dnz_v)
        pltpu.sync_copy(rnx_h.at[pl.ds(base, b_per_w)], rnx_v)
        pltpu.sync_copy(rny_h.at[pl.ds(base, b_per_w)], rny_v)
        pltpu.sync_copy(rnz_h.at[pl.ds(base, b_per_w)], rnz_v)
        pltpu.sync_copy(idx_h.at[pl.ds(base, b_per_w)], idx_v)
        for c in range(n_chunks):
            sl = pl.ds(c * 16, 16)
            iv = idx_v[sl]
            dx = plsc.load_gather(dnx_v, [iv]) - rnx_v[sl]
            dy = plsc.load_gather(dny_v, [iv]) - rny_v[sl]
            dz = plsc.load_gather(dnz_v, [iv]) - rnz_v[sl]
            out_v[sl] = dx * dx + dy * dy + dz * dz
        pltpu.sync_copy(out_v, out_h.at[pl.ds(base, b_per_w)])

    return residual_kernel(*dn_planes, *rn_planes, idx)


def kernel(rec, data, rec_normals, data_normals):
    rec2 = rec[0]
    data2 = data[0]
    rn = rec_normals[0]
    dn = data_normals[0]

    data_t = data2.T

    idx = pl.pallas_call(
        _argmin_body,
        grid=(_N // _NT,),
        in_specs=[
            pl.BlockSpec((_NT, 3), lambda i: (i, 0)),
            pl.BlockSpec((3, _M), lambda i: (0, 0)),
        ],
        out_specs=pl.BlockSpec((_NT, 1), lambda i: (i, 0)),
        out_shape=jax.ShapeDtypeStruct((_N, 1), jnp.int32),
    )(rec2, data_t)
    idx = idx.reshape(_N)

    dn_planes = (dn[:, 0], dn[:, 1], dn[:, 2])
    rn_planes = (rn[:, 0], rn[:, 1], rn[:, 2])
    sq = _sc_residuals(dn_planes, rn_planes, idx)

    loss = pl.pallas_call(
        _loss_body,
        in_specs=[pl.BlockSpec((64, 128), lambda: (0, 0))],
        out_specs=pl.BlockSpec((1, 1), lambda: (0, 0)),
        out_shape=jax.ShapeDtypeStruct((1, 1), jnp.float32),
    )(sq.reshape(64, 128))
    return loss[0, 0]


# speedup vs baseline: 1.5949x; 1.2613x over previous
import functools

import jax
import jax.numpy as jnp
from jax import lax
from jax.experimental import pallas as pl
from jax.experimental.pallas import tpu as pltpu
from jax.experimental.pallas import tpu_sc as plsc

_N = 8192
_M = 8192
_NT = 256


def _argmin_body(rec4_ref, data4_ref, idx_ref):
    rec4 = rec4_ref[...]
    data4 = data4_ref[...]
    s = lax.dot_general(
        rec4, data4,
        dimension_numbers=(((1,), (1,)), ((), ())),
        preferred_element_type=jnp.float32,
    )
    idx = jnp.argmin(s, axis=1).astype(jnp.int32)
    idx_ref[...] = idx.reshape(_NT, 1)


def _loss_body(sq_ref, out_ref):
    sq = sq_ref[...]
    out_ref[...] = (jnp.sum(jnp.sqrt(sq)) / _N).reshape(1, 1)


def _sc_residuals(dn_t, rn_t, idx):
    info = plsc.get_sparse_core_info()
    nw = info.num_cores * info.num_subcores
    b_per_w = _N // nw
    n_chunks = b_per_w // 16
    mesh = plsc.VectorSubcoreMesh(core_axis_name="c", subcore_axis_name="s")

    @functools.partial(
        pl.kernel,
        out_type=jax.ShapeDtypeStruct((_N,), jnp.float32),
        mesh=mesh,
        compiler_params=pltpu.CompilerParams(needs_layout_passes=False),
        scratch_types=[
            pltpu.VMEM((_M,), jnp.float32),
            pltpu.VMEM((_M,), jnp.float32),
            pltpu.VMEM((_M,), jnp.float32),
            pltpu.VMEM((b_per_w,), jnp.float32),
            pltpu.VMEM((b_per_w,), jnp.float32),
            pltpu.VMEM((b_per_w,), jnp.float32),
            pltpu.VMEM((b_per_w,), jnp.int32),
            pltpu.VMEM((b_per_w,), jnp.float32),
        ],
    )
    def residual_kernel(dnx_h, dny_h, dnz_h, rnx_h, rny_h, rnz_h, idx_h,
                        out_h, dnx_v, dny_v, dnz_v, rnx_v, rny_v, rnz_v,
                        idx_v, out_v):
        wid = lax.axis_index("s") * info.num_cores + lax.axis_index("c")
        base = wid * b_per_w
        pltpu.sync_copy(dnx_h, dnx_v)
        pltpu.sync_copy(dny_h, dny_v)
        pltpu.sync_copy(dnz_h, dnz_v)
        pltpu.sync_copy(rnx_h.at[pl.ds(base, b_per_w)], rnx_v)
        pltpu.sync_copy(rny_h.at[pl.ds(base, b_per_w)], rny_v)
        pltpu.sync_copy(rnz_h.at[pl.ds(base, b_per_w)], rnz_v)
        pltpu.sync_copy(idx_h.at[pl.ds(base, b_per_w)], idx_v)
        for c in range(n_chunks):
            sl = pl.ds(c * 16, 16)
            iv = idx_v[sl]
            dx = plsc.load_gather(dnx_v, [iv]) - rnx_v[sl]
            dy = plsc.load_gather(dny_v, [iv]) - rny_v[sl]
            dz = plsc.load_gather(dnz_v, [iv]) - rnz_v[sl]
            out_v[sl] = dx * dx + dy * dy + dz * dz
        pltpu.sync_copy(out_v, out_h.at[pl.ds(base, b_per_w)])

    return residual_kernel(*dn_t, *rn_t, idx)


def kernel(rec, data, rec_normals, data_normals):
    rec2 = rec[0]
    data2 = data[0]
    rn = rec_normals[0]
    dn = data_normals[0]

    rec4 = jnp.concatenate(
        [rec2, jnp.ones((_N, 1), jnp.float32)], axis=1)
    y2 = jnp.sum(data2 * data2, axis=1, keepdims=True)
    data4 = jnp.concatenate([-2.0 * data2, y2], axis=1)

    idx = pl.pallas_call(
        _argmin_body,
        grid=(_N // _NT,),
        in_specs=[
            pl.BlockSpec((_NT, 4), lambda i: (i, 0)),
            pl.BlockSpec((_M, 4), lambda i: (0, 0)),
        ],
        out_specs=pl.BlockSpec((_NT, 1), lambda i: (i, 0)),
        out_shape=jax.ShapeDtypeStruct((_N, 1), jnp.int32),
    )(rec4, data4)
    idx = idx.reshape(_N)

    sq = _sc_residuals((dn[:, 0], dn[:, 1], dn[:, 2]),
                       (rn[:, 0], rn[:, 1], rn[:, 2]), idx)

    loss = pl.pallas_call(
        _loss_body,
        in_specs=[pl.BlockSpec((64, 128), lambda: (0, 0))],
        out_specs=pl.BlockSpec((1, 1), lambda: (0, 0)),
        out_shape=jax.ShapeDtypeStruct((1, 1), jnp.float32),
    )(sq.reshape(64, 128))
    return loss[0, 0]


# speedup vs baseline: 1.6152x; 1.0127x over previous
import functools

import jax
import jax.numpy as jnp
from jax import lax
from jax.experimental import pallas as pl
from jax.experimental.pallas import tpu as pltpu
from jax.experimental.pallas import tpu_sc as plsc

_N = 8192
_M = 8192
_NT = 256


def _argmin_body(rec_ref, data_ref, idx_ref, data4_ref):
    @pl.when(pl.program_id(0) == 0)
    def _build_data4():
        d = data_ref[...]
        y2 = jnp.sum(d * d, axis=1, keepdims=True)
        data4_ref[...] = jnp.concatenate([-2.0 * d, y2], axis=1)

    rec = rec_ref[...]
    rec4 = jnp.concatenate(
        [rec, jnp.ones((_NT, 1), jnp.float32)], axis=1)
    s = lax.dot_general(
        rec4, data4_ref[...],
        dimension_numbers=(((1,), (1,)), ((), ())),
        preferred_element_type=jnp.float32,
    )
    idx_ref[...] = jnp.argmin(s, axis=1).astype(jnp.int32)


def _loss_body(sq_ref, out_ref):
    sq = sq_ref[...]
    out_ref[...] = (jnp.sum(jnp.sqrt(sq)) / _N).reshape(1, 1)


def _sc_residuals(dn_t, rn_t, idx):
    info = plsc.get_sparse_core_info()
    nw = info.num_cores * info.num_subcores
    b_per_w = _N // nw
    n_chunks = b_per_w // 16
    mesh = plsc.VectorSubcoreMesh(core_axis_name="c", subcore_axis_name="s")

    @functools.partial(
        pl.kernel,
        out_type=jax.ShapeDtypeStruct((_N,), jnp.float32),
        mesh=mesh,
        compiler_params=pltpu.CompilerParams(needs_layout_passes=False),
        scratch_types=[
            pltpu.VMEM((_M,), jnp.float32),
            pltpu.VMEM((_M,), jnp.float32),
            pltpu.VMEM((_M,), jnp.float32),
            pltpu.VMEM((b_per_w,), jnp.float32),
            pltpu.VMEM((b_per_w,), jnp.float32),
            pltpu.VMEM((b_per_w,), jnp.float32),
            pltpu.VMEM((b_per_w,), jnp.int32),
            pltpu.VMEM((b_per_w,), jnp.float32),
        ],
    )
    def residual_kernel(dnx_h, dny_h, dnz_h, rnx_h, rny_h, rnz_h, idx_h,
                        out_h, dnx_v, dny_v, dnz_v, rnx_v, rny_v, rnz_v,
                        idx_v, out_v):
        wid = lax.axis_index("s") * info.num_cores + lax.axis_index("c")
        base = wid * b_per_w
        pltpu.sync_copy(dnx_h, dnx_v)
        pltpu.sync_copy(dny_h, dny_v)
        pltpu.sync_copy(dnz_h, dnz_v)
        pltpu.sync_copy(rnx_h.at[pl.ds(base, b_per_w)], rnx_v)
        pltpu.sync_copy(rny_h.at[pl.ds(base, b_per_w)], rny_v)
        pltpu.sync_copy(rnz_h.at[pl.ds(base, b_per_w)], rnz_v)
        pltpu.sync_copy(idx_h.at[pl.ds(base, b_per_w)], idx_v)
        for c in range(n_chunks):
            sl = pl.ds(c * 16, 16)
            iv = idx_v[sl]
            dx = plsc.load_gather(dnx_v, [iv]) - rnx_v[sl]
            dy = plsc.load_gather(dny_v, [iv]) - rny_v[sl]
            dz = plsc.load_gather(dnz_v, [iv]) - rnz_v[sl]
            out_v[sl] = dx * dx + dy * dy + dz * dz
        pltpu.sync_copy(out_v, out_h.at[pl.ds(base, b_per_w)])

    return residual_kernel(*dn_t, *rn_t, idx)


def kernel(rec, data, rec_normals, data_normals):
    rec2 = rec[0]
    data2 = data[0]
    rn = rec_normals[0]
    dn = data_normals[0]

    idx = pl.pallas_call(
        _argmin_body,
        grid=(_N // _NT,),
        in_specs=[
            pl.BlockSpec((_NT, 3), lambda i: (i, 0)),
            pl.BlockSpec((_M, 3), lambda i: (0, 0)),
        ],
        out_specs=pl.BlockSpec((_NT,), lambda i: (i,)),
        out_shape=jax.ShapeDtypeStruct((_N,), jnp.int32),
        scratch_shapes=[pltpu.VMEM((_M, 4), jnp.float32)],
    )(rec2, data2)

    sq = _sc_residuals((dn[:, 0], dn[:, 1], dn[:, 2]),
                       (rn[:, 0], rn[:, 1], rn[:, 2]), idx)

    loss = pl.pallas_call(
        _loss_body,
        in_specs=[pl.BlockSpec((64, 128), lambda: (0, 0))],
        out_specs=pl.BlockSpec((1, 1), lambda: (0, 0)),
        out_shape=jax.ShapeDtypeStruct((1, 1), jnp.float32),
    )(sq.reshape(64, 128))
    return loss[0, 0]


# speedup vs baseline: 1.6992x; 1.0520x over previous
import functools

import jax
import jax.numpy as jnp
from jax import lax
from jax.experimental import pallas as pl
from jax.experimental.pallas import tpu as pltpu
from jax.experimental.pallas import tpu_sc as plsc

_N = 8192
_M = 8192
_NT = 512


def _argmin_body(rec_ref, data_ref, idx_ref, data4_ref):
    @pl.when(pl.program_id(0) == 0)
    def _build_data4():
        d = data_ref[...]
        y2 = jnp.sum(d * d, axis=1, keepdims=True)
        data4_ref[...] = jnp.concatenate([-2.0 * d, y2], axis=1)

    rec = rec_ref[...]
    rec4 = jnp.concatenate(
        [rec, jnp.ones((_NT, 1), jnp.float32)], axis=1)
    s = lax.dot_general(
        rec4, data4_ref[...],
        dimension_numbers=(((1,), (1,)), ((), ())),
        preferred_element_type=jnp.float32,
    )
    idx_ref[...] = jnp.argmin(s, axis=1).astype(jnp.int32)


def _loss_body(sq_ref, out_ref):
    sq = sq_ref[...]
    out_ref[...] = (jnp.sum(jnp.sqrt(sq)) / _N).reshape(1, 1)


def _sc_residuals(dn_t, rn_t, idx):
    info = plsc.get_sparse_core_info()
    nw = info.num_cores * info.num_subcores
    b_per_w = _N // nw
    n_chunks = b_per_w // 16
    mesh = plsc.VectorSubcoreMesh(core_axis_name="c", subcore_axis_name="s")

    @functools.partial(
        pl.kernel,
        out_type=jax.ShapeDtypeStruct((_N,), jnp.float32),
        mesh=mesh,
        compiler_params=pltpu.CompilerParams(needs_layout_passes=False),
        scratch_types=[
            pltpu.VMEM((_M,), jnp.float32),
            pltpu.VMEM((_M,), jnp.float32),
            pltpu.VMEM((_M,), jnp.float32),
            pltpu.VMEM((b_per_w,), jnp.float32),
            pltpu.VMEM((b_per_w,), jnp.float32),
            pltpu.VMEM((b_per_w,), jnp.float32),
            pltpu.VMEM((b_per_w,), jnp.int32),
            pltpu.VMEM((b_per_w,), jnp.float32),
        ],
    )
    def residual_kernel(dnx_h, dny_h, dnz_h, rnx_h, rny_h, rnz_h, idx_h,
                        out_h, dnx_v, dny_v, dnz_v, rnx_v, rny_v, rnz_v,
                        idx_v, out_v):
        wid = lax.axis_index("s") * info.num_cores + lax.axis_index("c")
        base = wid * b_per_w
        pltpu.sync_copy(dnx_h, dnx_v)
        pltpu.sync_copy(dny_h, dny_v)
        pltpu.sync_copy(dnz_h, dnz_v)
        pltpu.sync_copy(rnx_h.at[pl.ds(base, b_per_w)], rnx_v)
        pltpu.sync_copy(rny_h.at[pl.ds(base, b_per_w)], rny_v)
        pltpu.sync_copy(rnz_h.at[pl.ds(base, b_per_w)], rnz_v)
        pltpu.sync_copy(idx_h.at[pl.ds(base, b_per_w)], idx_v)
        for c in range(n_chunks):
            sl = pl.ds(c * 16, 16)
            iv = idx_v[sl]
            dx = plsc.load_gather(dnx_v, [iv]) - rnx_v[sl]
            dy = plsc.load_gather(dny_v, [iv]) - rny_v[sl]
            dz = plsc.load_gather(dnz_v, [iv]) - rnz_v[sl]
            out_v[sl] = dx * dx + dy * dy + dz * dz
        pltpu.sync_copy(out_v, out_h.at[pl.ds(base, b_per_w)])

    return residual_kernel(*dn_t, *rn_t, idx)


def kernel(rec, data, rec_normals, data_normals):
    rec2 = rec[0]
    data2 = data[0]
    rn = rec_normals[0]
    dn = data_normals[0]

    idx = pl.pallas_call(
        _argmin_body,
        grid=(_N // _NT,),
        in_specs=[
            pl.BlockSpec((_NT, 3), lambda i: (i, 0)),
            pl.BlockSpec((_M, 3), lambda i: (0, 0)),
        ],
        out_specs=pl.BlockSpec((_NT,), lambda i: (i,)),
        out_shape=jax.ShapeDtypeStruct((_N,), jnp.int32),
        scratch_shapes=[pltpu.VMEM((_M, 4), jnp.float32)],
    )(rec2, data2)

    sq = _sc_residuals((dn[:, 0], dn[:, 1], dn[:, 2]),
                       (rn[:, 0], rn[:, 1], rn[:, 2]), idx)

    loss = pl.pallas_call(
        _loss_body,
        in_specs=[pl.BlockSpec((64, 128), lambda: (0, 0))],
        out_specs=pl.BlockSpec((1, 1), lambda: (0, 0)),
        out_shape=jax.ShapeDtypeStruct((1, 1), jnp.float32),
    )(sq.reshape(64, 128))
    return loss[0, 0]


# speedup vs baseline: 1.7102x; 1.0065x over previous
import functools

import jax
import jax.numpy as jnp
from jax import lax
from jax.experimental import pallas as pl
from jax.experimental.pallas import tpu as pltpu
from jax.experimental.pallas import tpu_sc as plsc

_N = 8192
_M = 8192
_NT = 1024


def _argmin_body(rec_ref, data_ref, idx_ref, data4_ref):
    @pl.when(pl.program_id(0) == 0)
    def _build_data4():
        d = data_ref[...]
        y2 = jnp.sum(d * d, axis=1, keepdims=True)
        data4_ref[...] = jnp.concatenate([-2.0 * d, y2], axis=1)

    rec = rec_ref[...]
    rec4 = jnp.concatenate(
        [rec, jnp.ones((_NT, 1), jnp.float32)], axis=1)
    s = lax.dot_general(
        rec4, data4_ref[...],
        dimension_numbers=(((1,), (1,)), ((), ())),
        preferred_element_type=jnp.float32,
    )
    idx_ref[...] = jnp.argmin(s, axis=1).astype(jnp.int32)


def _loss_body(sq_ref, out_ref):
    sq = sq_ref[...]
    out_ref[...] = (jnp.sum(jnp.sqrt(sq)) / _N).reshape(1, 1)


def _sc_residuals(dn_t, rn_t, idx):
    info = plsc.get_sparse_core_info()
    nw = info.num_cores * info.num_subcores
    b_per_w = _N // nw
    n_chunks = b_per_w // 16
    mesh = plsc.VectorSubcoreMesh(core_axis_name="c", subcore_axis_name="s")

    @functools.partial(
        pl.kernel,
        out_type=jax.ShapeDtypeStruct((_N,), jnp.float32),
        mesh=mesh,
        compiler_params=pltpu.CompilerParams(needs_layout_passes=False),
        scratch_types=[
            pltpu.VMEM((_M,), jnp.float32),
            pltpu.VMEM((_M,), jnp.float32),
            pltpu.VMEM((_M,), jnp.float32),
            pltpu.VMEM((b_per_w,), jnp.float32),
            pltpu.VMEM((b_per_w,), jnp.float32),
            pltpu.VMEM((b_per_w,), jnp.float32),
            pltpu.VMEM((b_per_w,), jnp.int32),
            pltpu.VMEM((b_per_w,), jnp.float32),
        ],
    )
    def residual_kernel(dnx_h, dny_h, dnz_h, rnx_h, rny_h, rnz_h, idx_h,
                        out_h, dnx_v, dny_v, dnz_v, rnx_v, rny_v, rnz_v,
                        idx_v, out_v):
        wid = lax.axis_index("s") * info.num_cores + lax.axis_index("c")
        base = wid * b_per_w
        pltpu.sync_copy(dnx_h, dnx_v)
        pltpu.sync_copy(dny_h, dny_v)
        pltpu.sync_copy(dnz_h, dnz_v)
        pltpu.sync_copy(rnx_h.at[pl.ds(base, b_per_w)], rnx_v)
        pltpu.sync_copy(rny_h.at[pl.ds(base, b_per_w)], rny_v)
        pltpu.sync_copy(rnz_h.at[pl.ds(base, b_per_w)], rnz_v)
        pltpu.sync_copy(idx_h.at[pl.ds(base, b_per_w)], idx_v)
        for c in range(n_chunks):
            sl = pl.ds(c * 16, 16)
            iv = idx_v[sl]
            dx = plsc.load_gather(dnx_v, [iv]) - rnx_v[sl]
            dy = plsc.load_gather(dny_v, [iv]) - rny_v[sl]
            dz = plsc.load_gather(dnz_v, [iv]) - rnz_v[sl]
            out_v[sl] = dx * dx + dy * dy + dz * dz
        pltpu.sync_copy(out_v, out_h.at[pl.ds(base, b_per_w)])

    return residual_kernel(*dn_t, *rn_t, idx)


def kernel(rec, data, rec_normals, data_normals):
    rec2 = rec[0]
    data2 = data[0]
    rn = rec_normals[0]
    dn = data_normals[0]

    idx = pl.pallas_call(
        _argmin_body,
        grid=(_N // _NT,),
        in_specs=[
            pl.BlockSpec((_NT, 3), lambda i: (i, 0)),
            pl.BlockSpec((_M, 3), lambda i: (0, 0)),
        ],
        out_specs=pl.BlockSpec((_NT,), lambda i: (i,)),
        out_shape=jax.ShapeDtypeStruct((_N,), jnp.int32),
        scratch_shapes=[pltpu.VMEM((_M, 4), jnp.float32)],
    )(rec2, data2)

    sq = _sc_residuals((dn[:, 0], dn[:, 1], dn[:, 2]),
                       (rn[:, 0], rn[:, 1], rn[:, 2]), idx)

    loss = pl.pallas_call(
        _loss_body,
        in_specs=[pl.BlockSpec((64, 128), lambda: (0, 0))],
        out_specs=pl.BlockSpec((1, 1), lambda: (0, 0)),
        out_shape=jax.ShapeDtypeStruct((1, 1), jnp.float32),
    )(sq.reshape(64, 128))
    return loss[0, 0]
